# Initial kernel scaffold; baseline (speedup 1.0000x reference)
import jax
import jax.numpy as jnp
from jax.experimental import pallas as pl


def kernel(text_cache, prompt_ids):
    raise NotImplementedError("write your pallas kernel here")


# speedup vs baseline: 1.5286x; 1.5286x over previous
import functools

import jax
import jax.numpy as jnp
from jax import lax
from jax.experimental import pallas as pl
from jax.experimental.pallas import tpu as pltpu
from jax.experimental.pallas import tpu_sc as plsc

_D = 512
_B = 16384

_info = plsc.get_sparse_core_info()
_NC = _info.num_cores
_NS = _info.num_subcores
_NW = _NC * _NS
_BPW = _B // _NW
_C = 64
_NCHUNK = _BPW // _C

_mesh = plsc.VectorSubcoreMesh(core_axis_name="c", subcore_axis_name="s")


@functools.partial(
    pl.kernel,
    mesh=_mesh,
    out_type=jax.ShapeDtypeStruct((_B, _D), jnp.float32),
    scratch_types=[
        pltpu.VMEM((_BPW,), jnp.int32),
        pltpu.VMEM((2, _C, _D), jnp.float32),
        pltpu.SemaphoreType.DMA,
        pltpu.SemaphoreType.DMA,
    ],
)
def _gather_rows(table_hbm, idx_hbm, out_hbm, idx_v, rows_v, sem0, sem1):
    wid = lax.axis_index("s") * _NC + lax.axis_index("c")
    base = wid * _BPW
    pltpu.sync_copy(idx_hbm.at[pl.ds(base, _BPW)], idx_v)
    sems = (sem0, sem1)
    pltpu.async_copy(
        table_hbm.at[idx_v.at[pl.ds(0, _C)]], rows_v.at[0], sems[0]
    )
    for j in range(_NCHUNK):
        cur = j % 2
        nxt = (j + 1) % 2
        if j + 1 < _NCHUNK:
            pltpu.async_copy(
                table_hbm.at[idx_v.at[pl.ds((j + 1) * _C, _C)]],
                rows_v.at[nxt],
                sems[nxt],
            )
        pltpu.make_async_copy(
            table_hbm.at[idx_v.at[pl.ds(j * _C, _C)]], rows_v.at[cur], sems[cur]
        ).wait()
        pltpu.sync_copy(rows_v.at[cur], out_hbm.at[pl.ds(base + j * _C, _C)])


def kernel(text_cache, prompt_ids):
    return _gather_rows(text_cache, prompt_ids.astype(jnp.int32))


# speedup vs baseline: 1.5571x; 1.0186x over previous
import functools

import jax
import jax.numpy as jnp
from jax import lax
from jax.experimental import pallas as pl
from jax.experimental.pallas import tpu as pltpu
from jax.experimental.pallas import tpu_sc as plsc

_D = 512
_B = 16384

_info = plsc.get_sparse_core_info()
_NC = _info.num_cores
_NS = _info.num_subcores
_NW = _NC * _NS
_BPW = _B // _NW
_C = 64
_NCHUNK = _BPW // _C
_NB = 3

_mesh = plsc.VectorSubcoreMesh(core_axis_name="c", subcore_axis_name="s")


@functools.partial(
    pl.kernel,
    mesh=_mesh,
    out_type=jax.ShapeDtypeStruct((_B, _D), jnp.float32),
    scratch_types=[
        pltpu.VMEM((_BPW,), jnp.int32),
        pltpu.VMEM((_NB, _C, _D), jnp.float32),
        [pltpu.SemaphoreType.DMA] * _NB,
        [pltpu.SemaphoreType.DMA] * _NB,
    ],
)
def _gather_rows(table_hbm, idx_hbm, out_hbm, idx_v, rows_v, gsems, ssems):
    wid = lax.axis_index("s") * _NC + lax.axis_index("c")
    base = wid * _BPW
    pltpu.sync_copy(idx_hbm.at[pl.ds(base, _BPW)], idx_v)

    def gather(j, b):
        return pltpu.make_async_copy(
            table_hbm.at[idx_v.at[pl.ds(j * _C, _C)]], rows_v.at[b], gsems[b]
        )

    def writeback(j, b):
        return pltpu.make_async_copy(
            rows_v.at[b], out_hbm.at[pl.ds(base + j * _C, _C)], ssems[b]
        )

    for b in range(_NB):
        gather(b, b).start()
    for j in range(_NCHUNK):
        b = j % _NB
        gather(j, b).wait()
        writeback(j, b).start()
        writeback(j, b).wait()
        nxt = j + _NB
        if nxt < _NCHUNK:
            gather(nxt, b).start()


def kernel(text_cache, prompt_ids):
    return _gather_rows(text_cache, prompt_ids.astype(jnp.int32))
